# baseline (speedup 1.0000x reference)
import functools

import jax
import jax.numpy as jnp
from jax.experimental import pallas as pl
from jax.experimental.pallas import tpu as pltpu

N = 10000
DIN = 128
DOUT = 16
TILE = 200
NT = N // TILE


def _sgc_kernel(x_ref, w_ref, b_ref, g1_ref, g2_ref, o_ref, h0_ref, h1_ref):
    k = pl.program_id(0)
    i = pl.program_id(1)
    even = (i % 2) == 0

    @pl.when((k == 0) & (i == 0))
    def _prologue():
        h0_ref[...] = jax.nn.relu(
            jnp.dot(x_ref[...], w_ref[...], preferred_element_type=jnp.float32)
            + b_ref[...]
        )

    @pl.when((k == 0) & even)
    def _p1e():
        t = jnp.dot(g1_ref[...], h0_ref[...], preferred_element_type=jnp.float32)
        h1_ref[pl.ds(i * TILE, TILE), :] = t
        o_ref[...] = t

    @pl.when((k == 0) & ~even)
    def _p1o():
        t = jnp.dot(g2_ref[...], h0_ref[...], preferred_element_type=jnp.float32)
        h1_ref[pl.ds(i * TILE, TILE), :] = t
        o_ref[...] = t

    @pl.when((k == 1) & even)
    def _p2e():
        o_ref[...] = jnp.dot(
            g1_ref[...], h1_ref[...], preferred_element_type=jnp.float32
        )

    @pl.when((k == 1) & ~even)
    def _p2o():
        o_ref[...] = jnp.dot(
            g2_ref[...], h1_ref[...], preferred_element_type=jnp.float32
        )


@functools.partial(jax.jit, static_argnames=())
def kernel(x, g, W, b):
    b2 = b.reshape(1, DOUT)
    return pl.pallas_call(
        _sgc_kernel,
        grid=(2, NT),
        in_specs=[
            pl.BlockSpec((N, DIN), lambda k, i: (0, 0)),
            pl.BlockSpec((DIN, DOUT), lambda k, i: (0, 0)),
            pl.BlockSpec((1, DOUT), lambda k, i: (0, 0)),
            pl.BlockSpec((TILE, N), lambda k, i: (2 * (i // 2), 0)),
            pl.BlockSpec((TILE, N), lambda k, i: (2 * (i // 2) + 1, 0)),
        ],
        out_specs=pl.BlockSpec((TILE, DOUT), lambda k, i: (i, 0)),
        out_shape=jax.ShapeDtypeStruct((N, DOUT), jnp.float32),
        scratch_shapes=[
            pltpu.VMEM((N, DOUT), jnp.float32),
            pltpu.VMEM((N, DOUT), jnp.float32),
        ],
        compiler_params=pltpu.CompilerParams(
            dimension_semantics=("arbitrary", "arbitrary"),
            vmem_limit_bytes=120 * 1024 * 1024,
        ),
    )(x, W, b2, g, g)


# speedup vs baseline: 1.4502x; 1.4502x over previous
import functools

import jax
import jax.numpy as jnp
from jax.experimental import pallas as pl
from jax.experimental.pallas import tpu as pltpu

N = 10000
DIN = 128
DOUT = 16
TILE = 400
NT = N // TILE


def _sgc_kernel(x_ref, w_ref, b_ref, g_ref, o_ref, h0_ref, h1_ref):
    k = pl.program_id(0)
    i = pl.program_id(1)

    @pl.when((k == 0) & (i == 0))
    def _prologue():
        h0_ref[...] = jax.nn.relu(
            jnp.dot(x_ref[...], w_ref[...], preferred_element_type=jnp.float32)
            + b_ref[...]
        )

    @pl.when(k == 0)
    def _pass1():
        t = jnp.dot(g_ref[...], h0_ref[...], preferred_element_type=jnp.float32)
        h1_ref[pl.ds(i * TILE, TILE), :] = t
        o_ref[...] = t

    @pl.when(k == 1)
    def _pass2():
        o_ref[...] = jnp.dot(
            g_ref[...], h1_ref[...], preferred_element_type=jnp.float32
        )


@functools.partial(jax.jit, static_argnames=())
def kernel(x, g, W, b):
    b2 = b.reshape(1, DOUT)
    return pl.pallas_call(
        _sgc_kernel,
        grid=(2, NT),
        in_specs=[
            pl.BlockSpec((N, DIN), lambda k, i: (0, 0)),
            pl.BlockSpec((DIN, DOUT), lambda k, i: (0, 0)),
            pl.BlockSpec((1, DOUT), lambda k, i: (0, 0)),
            pl.BlockSpec((TILE, N), lambda k, i: (i, 0)),
        ],
        out_specs=pl.BlockSpec((TILE, DOUT), lambda k, i: (i, 0)),
        out_shape=jax.ShapeDtypeStruct((N, DOUT), jnp.float32),
        scratch_shapes=[
            pltpu.VMEM((N, DOUT), jnp.float32),
            pltpu.VMEM((N, DOUT), jnp.float32),
        ],
        compiler_params=pltpu.CompilerParams(
            dimension_semantics=("arbitrary", "arbitrary"),
            vmem_limit_bytes=120 * 1024 * 1024,
        ),
    )(x, W, b2, g)
